# Initial kernel scaffold; baseline (speedup 1.0000x reference)
import jax
import jax.numpy as jnp
from jax import lax
from jax.experimental import pallas as pl
from jax.experimental.pallas import tpu as pltpu
from jax.experimental.pallas import tpu_sc as plsc

N = 10000
E = 320000
C = 128
CP = C + 3
XPAD = 144
NPT = 320
NPAD = 32 * NPT

_info = plsc.get_sparse_core_info()
NC = _info.num_cores
NS = _info.num_subcores
NW = NC * NS

EPT = E // NW
WA = 200
WH = 2000
BH = 64


def _wid():
    return lax.axis_index("s") * NC + lax.axis_index("c")


def _pass_a_body(xp, idxi, idxj, s1_out, s2_out, bi, bj, ri, rj, acc, sem1, sem2):
    w = _wid()
    base = w * EPT

    def window(wi, carry):
        s1, s2 = carry
        off = base + wi * WA
        pltpu.sync_copy(idxi.at[pl.ds(off, WA)], bi)
        pltpu.sync_copy(idxj.at[pl.ds(off, WA)], bj)
        cp1 = pltpu.async_copy(xp.at[bi], ri, sem1)
        cp2 = pltpu.async_copy(xp.at[bj], rj, sem2)
        cp1.wait()
        cp2.wait()

        def edge(e, carry2):
            s1e, s2e = carry2
            for c8 in range(XPAD // 16):
                a = ri[e, pl.ds(c8 * 16, 16)]
                b = rj[e, pl.ds(c8 * 16, 16)]
                d = b - a
                s1e = s1e + d
                s2e = s2e + d * d
            return (s1e, s2e)

        return lax.fori_loop(0, WA, edge, (s1, s2))

    z = jnp.zeros((16,), jnp.float32)
    s1, s2 = lax.fori_loop(0, EPT // WA, window, (z, z))
    acc[pl.ds(0, 16)] = s1
    acc[pl.ds(16, 16)] = s2
    pltpu.sync_copy(acc.at[pl.ds(0, 16)], s1_out.at[w])
    pltpu.sync_copy(acc.at[pl.ds(16, 16)], s2_out.at[w])


def _run_pass_a(xp, idxi, idxj):
    mesh = plsc.VectorSubcoreMesh(core_axis_name="c", subcore_axis_name="s")
    f = pl.kernel(
        _pass_a_body,
        mesh=mesh,
        out_type=[
            jax.ShapeDtypeStruct((NW, 16), jnp.float32),
            jax.ShapeDtypeStruct((NW, 16), jnp.float32),
        ],
        scratch_types=[
            pltpu.VMEM((WA,), jnp.int32),
            pltpu.VMEM((WA,), jnp.int32),
            pltpu.VMEM((WA, XPAD), jnp.float32),
            pltpu.VMEM((WA, XPAD), jnp.float32),
            pltpu.VMEM((32,), jnp.float32),
            pltpu.SemaphoreType.DMA,
            pltpu.SemaphoreType.DMA,
        ],
    )
    return f(xp, idxi, idxj)


def _pass_b_body(u, v, idxi, idxj, hpre, st_out, bi, bj, ru, rv, hb, acc, sem1, sem2):
    w = _wid()
    base = w * EPT

    def window(wi, carry):
        off = base + wi * WA
        pltpu.sync_copy(idxi.at[pl.ds(off, WA)], bi)
        pltpu.sync_copy(idxj.at[pl.ds(off, WA)], bj)
        cp1 = pltpu.async_copy(u.at[bi], ru, sem1)
        cp2 = pltpu.async_copy(v.at[bj], rv, sem2)
        cp1.wait()
        cp2.wait()

        def edge(e, carry2):
            out = []
            for c8 in range(8):
                h = ru[e, pl.ds(c8 * 16, 16)] + rv[e, pl.ds(c8 * 16, 16)]
                hb[e, pl.ds(c8 * 16, 16)] = h
                s1 = carry2[2 * c8] + h
                s2 = carry2[2 * c8 + 1] + h * h
                out.append(s1)
                out.append(s2)
            return tuple(out)

        carry = lax.fori_loop(0, WA, edge, carry)
        pltpu.sync_copy(hb, hpre.at[pl.ds(off, WA)])
        return carry

    z = jnp.zeros((16,), jnp.float32)
    carry = tuple(z for _ in range(16))
    carry = lax.fori_loop(0, EPT // WA, window, carry)
    for c8 in range(8):
        acc[pl.ds(c8 * 16, 16)] = carry[2 * c8]
        acc[pl.ds(128 + c8 * 16, 16)] = carry[2 * c8 + 1]
    pltpu.sync_copy(acc, st_out.at[w])


def _run_pass_b(u, v, idxi, idxj):
    mesh = plsc.VectorSubcoreMesh(core_axis_name="c", subcore_axis_name="s")
    f = pl.kernel(
        _pass_b_body,
        mesh=mesh,
        out_type=[
            jax.ShapeDtypeStruct((E, C), jnp.float32),
            jax.ShapeDtypeStruct((NW, 2 * C), jnp.float32),
        ],
        scratch_types=[
            pltpu.VMEM((WA,), jnp.int32),
            pltpu.VMEM((WA,), jnp.int32),
            pltpu.VMEM((WA, C), jnp.float32),
            pltpu.VMEM((WA, C), jnp.float32),
            pltpu.VMEM((WA, C), jnp.float32),
            pltpu.VMEM((2 * C,), jnp.float32),
            pltpu.SemaphoreType.DMA,
            pltpu.SemaphoreType.DMA,
        ],
    )
    return f(u, v, idxi, idxj)


def _pass_h_body(h2, idxi, out, ib, sel_id, sel_dst, rows, mx, sm, cnt, sem):
    w = _wid()
    lo = w * NPT
    hi = lo + NPT
    zero = jnp.zeros((16,), jnp.float32)
    one = jnp.ones((16,), jnp.float32)

    def zinit(n, _):
        for c8 in range(8):
            mx[n, pl.ds(c8 * 16, 16)] = zero
            sm[n, pl.ds(c8 * 16, 16)] = zero
        cnt[n, pl.ds(0, 16)] = zero
        return 0

    lax.fori_loop(0, NPT, zinit, 0)

    def window(wi, _):
        off = wi * WH
        pltpu.sync_copy(idxi.at[pl.ds(off, WH)], ib)

        def grp(g, fill):
            vv = ib[pl.ds(g * 16, 16)]
            m = (vv >= lo) & (vv < hi)
            ids = lax.iota(jnp.int32, 16) + (off + g * 16)
            plsc.store_compressed(sel_id.at[pl.ds(fill, 16)], ids, m)
            plsc.store_compressed(sel_dst.at[pl.ds(fill, 16)], vv, m)
            return fill + plsc.all_reduce_population_count(m)[0]

        fill = lax.fori_loop(0, WH // 16, grp, jnp.int32(0))

        nch = (fill + (BH - 1)) // BH

        def chunk(ci, _):
            cb = ci * BH
            pltpu.async_copy(h2.at[sel_id.at[pl.ds(cb, BH)]], rows, sem).wait()
            nk = jnp.minimum(fill - cb, BH)

            def edge(k, _2):
                d = sel_dst[cb + k]
                dl = d - lo
                for c8 in range(8):
                    r = rows[k, pl.ds(c8 * 16, 16)]
                    a = mx[dl, pl.ds(c8 * 16, 16)]
                    mx[dl, pl.ds(c8 * 16, 16)] = jnp.maximum(a, r)
                    s = sm[dl, pl.ds(c8 * 16, 16)]
                    sm[dl, pl.ds(c8 * 16, 16)] = s + r
                cnt[dl, pl.ds(0, 16)] = cnt[dl, pl.ds(0, 16)] + one
                return 0

            lax.fori_loop(0, nk, edge, 0)
            return 0

        lax.fori_loop(0, nch, chunk, 0)
        return 0

    lax.fori_loop(0, E // WH, window, 0)

    def finish(n, _):
        cv = cnt[n, pl.ds(0, 16)]
        den = jnp.maximum(cv, 1.0)
        for c8 in range(8):
            s = sm[n, pl.ds(c8 * 16, 16)]
            m = mx[n, pl.ds(c8 * 16, 16)]
            mx[n, pl.ds(c8 * 16, 16)] = m + s / den
        return 0

    lax.fori_loop(0, NPT, finish, 0)
    pltpu.sync_copy(mx, out.at[pl.ds(lo, NPT)])


def _run_pass_h(h2, idxi):
    mesh = plsc.VectorSubcoreMesh(core_axis_name="c", subcore_axis_name="s")
    f = pl.kernel(
        _pass_h_body,
        mesh=mesh,
        out_type=jax.ShapeDtypeStruct((NPAD, C), jnp.float32),
        scratch_types=[
            pltpu.VMEM((WH,), jnp.int32),
            pltpu.VMEM((WH + 16,), jnp.int32),
            pltpu.VMEM((WH + 16,), jnp.int32),
            pltpu.VMEM((BH, C), jnp.float32),
            pltpu.VMEM((NPT, C), jnp.float32),
            pltpu.VMEM((NPT, C), jnp.float32),
            pltpu.VMEM((NPT, 16), jnp.float32),
            pltpu.SemaphoreType.DMA,
        ],
    )
    return f(h2, idxi)


TN = 640
TE = 4000


def _node_body(xp_ref, w_ref, cvec_ref, inv_ref, u_ref, v_ref):
    z = jnp.dot(xp_ref[...], w_ref[...], preferred_element_type=jnp.float32)
    zb = z[:, C:] * inv_ref[0, 0]
    u_ref[...] = z[:, :C] + cvec_ref[...] - zb
    v_ref[...] = zb


def _run_node(xp_pad, wcat, cvec, inv):
    grid = (NPAD // TN,)
    return pl.pallas_call(
        _node_body,
        grid=grid,
        in_specs=[
            pl.BlockSpec((TN, XPAD), lambda i: (i, 0)),
            pl.BlockSpec((XPAD, 2 * C), lambda i: (0, 0)),
            pl.BlockSpec((1, C), lambda i: (0, 0)),
            pl.BlockSpec((1, 1), lambda i: (0, 0), memory_space=pltpu.SMEM),
        ],
        out_specs=[
            pl.BlockSpec((TN, C), lambda i: (i, 0)),
            pl.BlockSpec((TN, C), lambda i: (i, 0)),
        ],
        out_shape=[
            jax.ShapeDtypeStruct((NPAD, C), jnp.float32),
            jax.ShapeDtypeStruct((NPAD, C), jnp.float32),
        ],
    )(xp_pad, wcat, cvec, inv)


def _lin_body(x_ref, a_ref, b_ref, w_ref, bias_ref, out_ref, st_ref, acc_ref):
    i = pl.program_id(0)
    t = jnp.maximum(x_ref[...] * a_ref[...] + b_ref[...], 0.0)
    m = jnp.dot(t, w_ref[...], preferred_element_type=jnp.float32) + bias_ref[...]
    out_ref[...] = m

    @pl.when(i == 0)
    def _():
        acc_ref[...] = jnp.zeros_like(acc_ref)

    acc_ref[0:1, :] += jnp.sum(m, axis=0, keepdims=True)
    acc_ref[1:2, :] += jnp.sum(m * m, axis=0, keepdims=True)

    @pl.when(i == pl.num_programs(0) - 1)
    def _():
        st_ref[...] = acc_ref[...]


def _run_lin(x, a, b, w, bias):
    grid = (E // TE,)
    return pl.pallas_call(
        _lin_body,
        grid=grid,
        in_specs=[
            pl.BlockSpec((TE, C), lambda i: (i, 0)),
            pl.BlockSpec((1, C), lambda i: (0, 0)),
            pl.BlockSpec((1, C), lambda i: (0, 0)),
            pl.BlockSpec((C, C), lambda i: (0, 0)),
            pl.BlockSpec((1, C), lambda i: (0, 0)),
        ],
        out_specs=[
            pl.BlockSpec((TE, C), lambda i: (i, 0)),
            pl.BlockSpec((2, C), lambda i: (0, 0)),
        ],
        out_shape=[
            jax.ShapeDtypeStruct((E, C), jnp.float32),
            jax.ShapeDtypeStruct((2, C), jnp.float32),
        ],
        scratch_shapes=[pltpu.VMEM((2, C), jnp.float32)],
    )(x, a, b, w, bias)


def _lin_res_body(x_ref, r_ref, a_ref, b_ref, ar_ref, br_ref, w_ref, bias_ref,
                  out_ref, st_ref, acc_ref):
    i = pl.program_id(0)
    h0 = jnp.maximum(r_ref[...] * ar_ref[...] + br_ref[...], 0.0)
    t = jnp.maximum(x_ref[...] * a_ref[...] + b_ref[...] + h0, 0.0)
    m = jnp.dot(t, w_ref[...], preferred_element_type=jnp.float32) + bias_ref[...]
    out_ref[...] = m

    @pl.when(i == 0)
    def _():
        acc_ref[...] = jnp.zeros_like(acc_ref)

    acc_ref[0:1, :] += jnp.sum(m, axis=0, keepdims=True)
    acc_ref[1:2, :] += jnp.sum(m * m, axis=0, keepdims=True)

    @pl.when(i == pl.num_programs(0) - 1)
    def _():
        st_ref[...] = acc_ref[...]


def _run_lin_res(x, r, a, b, ar, br, w, bias):
    grid = (E // TE,)
    return pl.pallas_call(
        _lin_res_body,
        grid=grid,
        in_specs=[
            pl.BlockSpec((TE, C), lambda i: (i, 0)),
            pl.BlockSpec((TE, C), lambda i: (i, 0)),
            pl.BlockSpec((1, C), lambda i: (0, 0)),
            pl.BlockSpec((1, C), lambda i: (0, 0)),
            pl.BlockSpec((1, C), lambda i: (0, 0)),
            pl.BlockSpec((1, C), lambda i: (0, 0)),
            pl.BlockSpec((C, C), lambda i: (0, 0)),
            pl.BlockSpec((1, C), lambda i: (0, 0)),
        ],
        out_specs=[
            pl.BlockSpec((TE, C), lambda i: (i, 0)),
            pl.BlockSpec((2, C), lambda i: (0, 0)),
        ],
        out_shape=[
            jax.ShapeDtypeStruct((E, C), jnp.float32),
            jax.ShapeDtypeStruct((2, C), jnp.float32),
        ],
        scratch_shapes=[pltpu.VMEM((2, C), jnp.float32)],
    )(x, r, a, b, ar, br, w, bias)


def _final_body(m4_ref, m2_ref, hp_ref, a4_ref, b4_ref, a2_ref, b2_ref,
                a0_ref, b0_ref, out_ref):
    h0 = jnp.maximum(hp_ref[...] * a0_ref[...] + b0_ref[...], 0.0)
    h1 = jnp.maximum(m2_ref[...] * a2_ref[...] + b2_ref[...] + h0, 0.0)
    out_ref[...] = jnp.maximum(m4_ref[...] * a4_ref[...] + b4_ref[...] + h1, 0.0)


def _run_final(m4, m2, hp, a4, b4, a2, b2, a0, b0):
    grid = (E // TE,)

    def vec():
        return pl.BlockSpec((1, C), lambda i: (0, 0))

    def big():
        return pl.BlockSpec((TE, C), lambda i: (i, 0))

    return pl.pallas_call(
        _final_body,
        grid=grid,
        in_specs=[big(), big(), big(), vec(), vec(), vec(), vec(), vec(), vec()],
        out_specs=big(),
        out_shape=jax.ShapeDtypeStruct((E, C), jnp.float32),
    )(m4, m2, hp, a4, b4, a2, b2, a0, b0)


def _bn_affine(st, g, beta):
    mu = st[0] / E
    var = st[1] / E - mu * mu
    a = g / jnp.sqrt(var + 1e-5)
    return (a[None, :], (beta - mu * a)[None, :])


def kernel(pos, x, edge_index, affine_w, affine_b, lin_W, lin_b, lin_g, lin_beta,
           rW1, rb1, rg1, rbe1, rW2, rb2, rgn, rbn):
    idx_i = edge_index[0]
    idx_j = edge_index[1]

    xf = jnp.concatenate([x, pos], axis=1)
    xp = jnp.pad(xf, ((0, 0), (0, XPAD - CP)))

    s1p, s2p = _run_pass_a(xp, idx_i, idx_j)
    s1 = jnp.sum(s1p)
    s2 = jnp.sum(s2p)
    m = E * CP
    var = (s2 - s1 * s1 / m) / (m - 1)
    sprime = jnp.sqrt(var) + 1e-5

    w_top = lin_W[:CP]
    w_bot = affine_w[:, None] * lin_W[CP:]
    wcat = jnp.pad(jnp.concatenate([w_top, w_bot], axis=1),
                   ((0, XPAD - CP), (0, 0)))
    c0 = affine_b @ lin_W[CP:] + lin_b
    xp_pad = jnp.pad(xp, ((0, NPAD - N), (0, 0)))
    inv = (1.0 / sprime).reshape(1, 1)
    u, v = _run_node(xp_pad, wcat, c0[None, :], inv)

    hpre, stp = _run_pass_b(u, v, idx_i, idx_j)
    st0 = jnp.sum(stp, axis=0)
    a0, b0 = _bn_affine(jnp.stack([st0[:C], st0[C:]]), lin_g, lin_beta)

    m1, st1 = _run_lin(hpre, a0, b0, rW1[0], rb1[0][None, :])
    a1, b1 = _bn_affine(st1, rg1[0], rbe1[0])
    m2, st2 = _run_lin(m1, a1, b1, rW2[0], rb2[0][None, :])
    a2, b2 = _bn_affine(st2, rgn[0], rbn[0])
    m3, st3 = _run_lin_res(m2, hpre, a2, b2, a0, b0, rW1[1], rb1[1][None, :])
    a3, b3 = _bn_affine(st3, rg1[1], rbe1[1])
    m4, st4 = _run_lin(m3, a3, b3, rW2[1], rb2[1][None, :])
    a4, b4 = _bn_affine(st4, rgn[1], rbn[1])
    h2 = _run_final(m4, m2, hpre, a4, b4, a2, b2, a0, b0)

    out = _run_pass_h(h2, idx_i)
    return out[:N]


# speedup vs baseline: 2.3483x; 2.3483x over previous
import jax
import jax.numpy as jnp
from jax import lax
from jax.experimental import pallas as pl
from jax.experimental.pallas import tpu as pltpu
from jax.experimental.pallas import tpu_sc as plsc

N = 10000
E = 320000
C = 128
CP = C + 3
XPAD = 144
NPT = 320
NPAD = 32 * NPT

_info = plsc.get_sparse_core_info()
NC = _info.num_cores
NS = _info.num_subcores
NW = NC * NS

EPT = E // NW
WA = 200
WH = 800
BH = 16
SELB = 1024


def _wid():
    return lax.axis_index("s") * NC + lax.axis_index("c")


def _pass_a_body(xp, idxi, idxj, s1_out, s2_out, bi, bj, ri, rj, acc, sem1, sem2):
    w = _wid()
    base = w * EPT

    def window(wi, carry):
        s1, s2 = carry
        off = base + wi * WA
        pltpu.sync_copy(idxi.at[pl.ds(off, WA)], bi)
        pltpu.sync_copy(idxj.at[pl.ds(off, WA)], bj)
        cp1 = pltpu.async_copy(xp.at[bi], ri, sem1)
        cp2 = pltpu.async_copy(xp.at[bj], rj, sem2)
        cp1.wait()
        cp2.wait()

        def edge(e, carry2):
            s1e, s2e = carry2
            for c8 in range(XPAD // 16):
                a = ri[e, pl.ds(c8 * 16, 16)]
                b = rj[e, pl.ds(c8 * 16, 16)]
                d = b - a
                s1e = s1e + d
                s2e = s2e + d * d
            return (s1e, s2e)

        return lax.fori_loop(0, WA, edge, (s1, s2))

    z = jnp.zeros((16,), jnp.float32)
    s1, s2 = lax.fori_loop(0, EPT // WA, window, (z, z))
    acc[pl.ds(0, 16)] = s1
    acc[pl.ds(16, 16)] = s2
    pltpu.sync_copy(acc.at[pl.ds(0, 16)], s1_out.at[w])
    pltpu.sync_copy(acc.at[pl.ds(16, 16)], s2_out.at[w])


def _run_pass_a(xp, idxi, idxj):
    mesh = plsc.VectorSubcoreMesh(core_axis_name="c", subcore_axis_name="s")
    f = pl.kernel(
        _pass_a_body,
        mesh=mesh,
        compiler_params=pltpu.CompilerParams(use_tc_tiling_on_sc=False, needs_layout_passes=False),
        out_type=[
            jax.ShapeDtypeStruct((NW, 16), jnp.float32),
            jax.ShapeDtypeStruct((NW, 16), jnp.float32),
        ],
        scratch_types=[
            pltpu.VMEM((WA,), jnp.int32),
            pltpu.VMEM((WA,), jnp.int32),
            pltpu.VMEM((WA, XPAD), jnp.float32),
            pltpu.VMEM((WA, XPAD), jnp.float32),
            pltpu.VMEM((32,), jnp.float32),
            pltpu.SemaphoreType.DMA,
            pltpu.SemaphoreType.DMA,
        ],
    )
    return f(xp, idxi, idxj)


def _pass_b_body(u, v, idxi, idxj, hpre, st_out, bi, bj, ru, rv, hb, acc, sem1, sem2):
    w = _wid()
    base = w * EPT

    def window(wi, carry):
        off = base + wi * WA
        pltpu.sync_copy(idxi.at[pl.ds(off, WA)], bi)
        pltpu.sync_copy(idxj.at[pl.ds(off, WA)], bj)
        cp1 = pltpu.async_copy(u.at[bi], ru, sem1)
        cp2 = pltpu.async_copy(v.at[bj], rv, sem2)
        cp1.wait()
        cp2.wait()

        def edge(e, carry2):
            out = []
            for c8 in range(8):
                h = ru[e, pl.ds(c8 * 16, 16)] + rv[e, pl.ds(c8 * 16, 16)]
                hb[e, pl.ds(c8 * 16, 16)] = h
                s1 = carry2[2 * c8] + h
                s2 = carry2[2 * c8 + 1] + h * h
                out.append(s1)
                out.append(s2)
            return tuple(out)

        carry = lax.fori_loop(0, WA, edge, carry)
        pltpu.sync_copy(hb, hpre.at[pl.ds(off, WA)])
        return carry

    z = jnp.zeros((16,), jnp.float32)
    carry = tuple(z for _ in range(16))
    carry = lax.fori_loop(0, EPT // WA, window, carry)
    for c8 in range(8):
        acc[pl.ds(c8 * 16, 16)] = carry[2 * c8]
        acc[pl.ds(128 + c8 * 16, 16)] = carry[2 * c8 + 1]
    pltpu.sync_copy(acc, st_out.at[w])


def _run_pass_b(u, v, idxi, idxj):
    mesh = plsc.VectorSubcoreMesh(core_axis_name="c", subcore_axis_name="s")
    f = pl.kernel(
        _pass_b_body,
        mesh=mesh,
        out_type=[
            jax.ShapeDtypeStruct((E, C), jnp.float32),
            jax.ShapeDtypeStruct((NW, 2 * C), jnp.float32),
        ],
        scratch_types=[
            pltpu.VMEM((WA,), jnp.int32),
            pltpu.VMEM((WA,), jnp.int32),
            pltpu.VMEM((WA, C), jnp.float32),
            pltpu.VMEM((WA, C), jnp.float32),
            pltpu.VMEM((WA, C), jnp.float32),
            pltpu.VMEM((2 * C,), jnp.float32),
            pltpu.SemaphoreType.DMA,
            pltpu.SemaphoreType.DMA,
        ],
    )
    return f(u, v, idxi, idxj)


def _pass_h_body(h2, idxi, out, ib, sel_id, sel_dst, rows, mx, sm, cnt, sem):
    w = _wid()
    lo = w * NPT
    hi = lo + NPT
    zero = jnp.zeros((16,), jnp.float32)
    one = jnp.ones((16,), jnp.float32)

    def zinit(n, _):
        for c8 in range(8):
            mx[n, pl.ds(c8 * 16, 16)] = zero
            sm[n, pl.ds(c8 * 16, 16)] = zero
        cnt[n, pl.ds(0, 16)] = zero
        return 0

    lax.fori_loop(0, NPT, zinit, 0)

    izero = jnp.zeros((16,), jnp.int32)

    def selinit(k, _):
        sel_id[pl.ds(k * 16, 16)] = izero
        sel_dst[pl.ds(k * 16, 16)] = izero
        return 0

    lax.fori_loop(0, SELB // 16, selinit, 0)

    def window(wi, _):
        off = wi * WH
        pltpu.sync_copy(idxi.at[pl.ds(off, WH)], ib)

        def grp(g, fill):
            vv = ib[pl.ds(g * 16, 16)]
            m = (vv >= lo) & (vv < hi)
            ids = lax.iota(jnp.int32, 16) + (off + g * 16)
            pref = plsc.cumsum(jnp.where(m, 1, 0))
            pos = fill + pref - 1
            plsc.store_scatter(sel_id, [pos], ids, mask=m)
            plsc.store_scatter(sel_dst, [pos], vv, mask=m)
            return fill + pref[15]

        fill = lax.fori_loop(0, WH // 16, grp, jnp.int32(0))

        nch = (fill + (BH - 1)) // BH

        def chunk(ci, _):
            cb = ci * BH
            idvec = sel_id[pl.ds(cb, BH)]
            pltpu.async_copy(h2.at[idvec], rows, sem).wait()
            nk = jnp.minimum(fill - cb, BH)

            def edge(k, _2):
                d = sel_dst[pl.ds(cb + k, 16)][0]
                dl = d - lo
                for c8 in range(8):
                    r = rows[k, pl.ds(c8 * 16, 16)]
                    a = mx[dl, pl.ds(c8 * 16, 16)]
                    mx[dl, pl.ds(c8 * 16, 16)] = jnp.maximum(a, r)
                    s = sm[dl, pl.ds(c8 * 16, 16)]
                    sm[dl, pl.ds(c8 * 16, 16)] = s + r
                cnt[dl, pl.ds(0, 16)] = cnt[dl, pl.ds(0, 16)] + one
                return 0

            lax.fori_loop(0, nk, edge, 0)
            return 0

        lax.fori_loop(0, nch, chunk, 0)
        return 0

    lax.fori_loop(0, E // WH, window, 0)

    def finish(n, _):
        cv = cnt[n, pl.ds(0, 16)]
        den = jnp.maximum(cv, 1.0)
        for c8 in range(8):
            s = sm[n, pl.ds(c8 * 16, 16)]
            m = mx[n, pl.ds(c8 * 16, 16)]
            mx[n, pl.ds(c8 * 16, 16)] = m + s / den
        return 0

    lax.fori_loop(0, NPT, finish, 0)
    pltpu.sync_copy(mx, out.at[pl.ds(lo, NPT)])


def _run_pass_h(h2, idxi):
    mesh = plsc.VectorSubcoreMesh(core_axis_name="c", subcore_axis_name="s")
    f = pl.kernel(
        _pass_h_body,
        mesh=mesh,
        compiler_params=pltpu.CompilerParams(needs_layout_passes=False),
        out_type=jax.ShapeDtypeStruct((NPAD, C), jnp.float32),
        scratch_types=[
            pltpu.VMEM((WH,), jnp.int32),
            pltpu.VMEM((SELB,), jnp.int32),
            pltpu.VMEM((SELB,), jnp.int32),
            pltpu.VMEM((BH, C), jnp.float32),
            pltpu.VMEM((NPT, C), jnp.float32),
            pltpu.VMEM((NPT, C), jnp.float32),
            pltpu.VMEM((NPT, 16), jnp.float32),
            pltpu.SemaphoreType.DMA,
        ],
    )
    return f(h2, idxi)


TN = 640
TE = 4000


def _node_body(xp_ref, w_ref, cvec_ref, inv_ref, u_ref, v_ref):
    z = jnp.dot(xp_ref[...], w_ref[...], preferred_element_type=jnp.float32)
    zb = z[:, C:] * inv_ref[0, 0]
    u_ref[...] = z[:, :C] + cvec_ref[...] - zb
    v_ref[...] = zb


def _run_node(xp_pad, wcat, cvec, inv):
    grid = (NPAD // TN,)
    return pl.pallas_call(
        _node_body,
        grid=grid,
        in_specs=[
            pl.BlockSpec((TN, XPAD), lambda i: (i, 0)),
            pl.BlockSpec((XPAD, 2 * C), lambda i: (0, 0)),
            pl.BlockSpec((1, C), lambda i: (0, 0)),
            pl.BlockSpec((1, 1), lambda i: (0, 0), memory_space=pltpu.SMEM),
        ],
        out_specs=[
            pl.BlockSpec((TN, C), lambda i: (i, 0)),
            pl.BlockSpec((TN, C), lambda i: (i, 0)),
        ],
        out_shape=[
            jax.ShapeDtypeStruct((NPAD, C), jnp.float32),
            jax.ShapeDtypeStruct((NPAD, C), jnp.float32),
        ],
    )(xp_pad, wcat, cvec, inv)


def _lin_body(x_ref, a_ref, b_ref, w_ref, bias_ref, out_ref, st_ref, acc_ref):
    i = pl.program_id(0)
    t = jnp.maximum(x_ref[...] * a_ref[...] + b_ref[...], 0.0)
    m = jnp.dot(t, w_ref[...], preferred_element_type=jnp.float32) + bias_ref[...]
    out_ref[...] = m

    @pl.when(i == 0)
    def _():
        acc_ref[...] = jnp.zeros_like(acc_ref)

    acc_ref[0:1, :] += jnp.sum(m, axis=0, keepdims=True)
    acc_ref[1:2, :] += jnp.sum(m * m, axis=0, keepdims=True)

    @pl.when(i == pl.num_programs(0) - 1)
    def _():
        st_ref[...] = acc_ref[...]


def _run_lin(x, a, b, w, bias):
    grid = (E // TE,)
    return pl.pallas_call(
        _lin_body,
        grid=grid,
        in_specs=[
            pl.BlockSpec((TE, C), lambda i: (i, 0)),
            pl.BlockSpec((1, C), lambda i: (0, 0)),
            pl.BlockSpec((1, C), lambda i: (0, 0)),
            pl.BlockSpec((C, C), lambda i: (0, 0)),
            pl.BlockSpec((1, C), lambda i: (0, 0)),
        ],
        out_specs=[
            pl.BlockSpec((TE, C), lambda i: (i, 0)),
            pl.BlockSpec((2, C), lambda i: (0, 0)),
        ],
        out_shape=[
            jax.ShapeDtypeStruct((E, C), jnp.float32),
            jax.ShapeDtypeStruct((2, C), jnp.float32),
        ],
        scratch_shapes=[pltpu.VMEM((2, C), jnp.float32)],
    )(x, a, b, w, bias)


def _lin_res_body(x_ref, r_ref, a_ref, b_ref, ar_ref, br_ref, w_ref, bias_ref,
                  out_ref, st_ref, acc_ref):
    i = pl.program_id(0)
    h0 = jnp.maximum(r_ref[...] * ar_ref[...] + br_ref[...], 0.0)
    t = jnp.maximum(x_ref[...] * a_ref[...] + b_ref[...] + h0, 0.0)
    m = jnp.dot(t, w_ref[...], preferred_element_type=jnp.float32) + bias_ref[...]
    out_ref[...] = m

    @pl.when(i == 0)
    def _():
        acc_ref[...] = jnp.zeros_like(acc_ref)

    acc_ref[0:1, :] += jnp.sum(m, axis=0, keepdims=True)
    acc_ref[1:2, :] += jnp.sum(m * m, axis=0, keepdims=True)

    @pl.when(i == pl.num_programs(0) - 1)
    def _():
        st_ref[...] = acc_ref[...]


def _run_lin_res(x, r, a, b, ar, br, w, bias):
    grid = (E // TE,)
    return pl.pallas_call(
        _lin_res_body,
        grid=grid,
        in_specs=[
            pl.BlockSpec((TE, C), lambda i: (i, 0)),
            pl.BlockSpec((TE, C), lambda i: (i, 0)),
            pl.BlockSpec((1, C), lambda i: (0, 0)),
            pl.BlockSpec((1, C), lambda i: (0, 0)),
            pl.BlockSpec((1, C), lambda i: (0, 0)),
            pl.BlockSpec((1, C), lambda i: (0, 0)),
            pl.BlockSpec((C, C), lambda i: (0, 0)),
            pl.BlockSpec((1, C), lambda i: (0, 0)),
        ],
        out_specs=[
            pl.BlockSpec((TE, C), lambda i: (i, 0)),
            pl.BlockSpec((2, C), lambda i: (0, 0)),
        ],
        out_shape=[
            jax.ShapeDtypeStruct((E, C), jnp.float32),
            jax.ShapeDtypeStruct((2, C), jnp.float32),
        ],
        scratch_shapes=[pltpu.VMEM((2, C), jnp.float32)],
    )(x, r, a, b, ar, br, w, bias)


def _final_body(m4_ref, m2_ref, hp_ref, a4_ref, b4_ref, a2_ref, b2_ref,
                a0_ref, b0_ref, out_ref):
    h0 = jnp.maximum(hp_ref[...] * a0_ref[...] + b0_ref[...], 0.0)
    h1 = jnp.maximum(m2_ref[...] * a2_ref[...] + b2_ref[...] + h0, 0.0)
    out_ref[...] = jnp.maximum(m4_ref[...] * a4_ref[...] + b4_ref[...] + h1, 0.0)


def _run_final(m4, m2, hp, a4, b4, a2, b2, a0, b0):
    grid = (E // TE,)

    def vec():
        return pl.BlockSpec((1, C), lambda i: (0, 0))

    def big():
        return pl.BlockSpec((TE, C), lambda i: (i, 0))

    return pl.pallas_call(
        _final_body,
        grid=grid,
        in_specs=[big(), big(), big(), vec(), vec(), vec(), vec(), vec(), vec()],
        out_specs=big(),
        out_shape=jax.ShapeDtypeStruct((E, C), jnp.float32),
    )(m4, m2, hp, a4, b4, a2, b2, a0, b0)


def _bn_affine(st, g, beta):
    mu = st[0] / E
    var = st[1] / E - mu * mu
    a = g / jnp.sqrt(var + 1e-5)
    return (a[None, :], (beta - mu * a)[None, :])


def kernel(pos, x, edge_index, affine_w, affine_b, lin_W, lin_b, lin_g, lin_beta,
           rW1, rb1, rg1, rbe1, rW2, rb2, rgn, rbn):
    idx_i = edge_index[0]
    idx_j = edge_index[1]

    xf = jnp.concatenate([x, pos], axis=1)
    xp = jnp.pad(xf, ((0, 0), (0, XPAD - CP)))

    s1p, s2p = _run_pass_a(xp, idx_i, idx_j)
    s1 = jnp.sum(s1p)
    s2 = jnp.sum(s2p)
    m = E * CP
    var = (s2 - s1 * s1 / m) / (m - 1)
    sprime = jnp.sqrt(var) + 1e-5

    w_top = lin_W[:CP]
    w_bot = affine_w[:, None] * lin_W[CP:]
    wcat = jnp.pad(jnp.concatenate([w_top, w_bot], axis=1),
                   ((0, XPAD - CP), (0, 0)))
    c0 = affine_b @ lin_W[CP:] + lin_b
    xp_pad = jnp.pad(xp, ((0, NPAD - N), (0, 0)))
    inv = (1.0 / sprime).reshape(1, 1)
    u, v = _run_node(xp_pad, wcat, c0[None, :], inv)

    hpre, stp = _run_pass_b(u, v, idx_i, idx_j)
    st0 = jnp.sum(stp, axis=0)
    a0, b0 = _bn_affine(jnp.stack([st0[:C], st0[C:]]), lin_g, lin_beta)

    m1, st1 = _run_lin(hpre, a0, b0, rW1[0], rb1[0][None, :])
    a1, b1 = _bn_affine(st1, rg1[0], rbe1[0])
    m2, st2 = _run_lin(m1, a1, b1, rW2[0], rb2[0][None, :])
    a2, b2 = _bn_affine(st2, rgn[0], rbn[0])
    m3, st3 = _run_lin_res(m2, hpre, a2, b2, a0, b0, rW1[1], rb1[1][None, :])
    a3, b3 = _bn_affine(st3, rg1[1], rbe1[1])
    m4, st4 = _run_lin(m3, a3, b3, rW2[1], rb2[1][None, :])
    a4, b4 = _bn_affine(st4, rgn[1], rbn[1])
    h2 = _run_final(m4, m2, hpre, a4, b4, a2, b2, a0, b0)

    out = _run_pass_h(h2, idx_i)
    return out[:N]


# speedup vs baseline: 2.5213x; 1.0736x over previous
import jax
import jax.numpy as jnp
from jax import lax
from jax.experimental import pallas as pl
from jax.experimental.pallas import tpu as pltpu
from jax.experimental.pallas import tpu_sc as plsc

N = 10000
E = 320000
C = 128
CP = C + 3
XPAD = 144
NPT = 320
NPAD = 32 * NPT

_info = plsc.get_sparse_core_info()
NC = _info.num_cores
NS = _info.num_subcores
NW = NC * NS

EPT = E // NW
WA = 200
WH = 800
NWIN = E // WH
BH = 16
SELB = 1024


def _wid():
    return lax.axis_index("s") * NC + lax.axis_index("c")


def _pass_a_body(xp, idxi, idxj, s1_out, s2_out, bi, bj, ri, rj, acc, sem1, sem2):
    w = _wid()
    base = w * EPT

    def window(wi, carry):
        s1, s2 = carry
        off = base + wi * WA
        pltpu.sync_copy(idxi.at[pl.ds(off, WA)], bi)
        pltpu.sync_copy(idxj.at[pl.ds(off, WA)], bj)
        cp1 = pltpu.async_copy(xp.at[bi], ri, sem1)
        cp2 = pltpu.async_copy(xp.at[bj], rj, sem2)
        cp1.wait()
        cp2.wait()

        def edge(e, carry2):
            s1e, s2e = carry2
            for c8 in range(XPAD // 16):
                a = ri[e, pl.ds(c8 * 16, 16)]
                b = rj[e, pl.ds(c8 * 16, 16)]
                d = b - a
                s1e = s1e + d
                s2e = s2e + d * d
            return (s1e, s2e)

        return lax.fori_loop(0, WA, edge, (s1, s2))

    z = jnp.zeros((16,), jnp.float32)
    s1, s2 = lax.fori_loop(0, EPT // WA, window, (z, z))
    acc[pl.ds(0, 16)] = s1
    acc[pl.ds(16, 16)] = s2
    pltpu.sync_copy(acc.at[pl.ds(0, 16)], s1_out.at[w])
    pltpu.sync_copy(acc.at[pl.ds(16, 16)], s2_out.at[w])


def _run_pass_a(xp, idxi, idxj):
    mesh = plsc.VectorSubcoreMesh(core_axis_name="c", subcore_axis_name="s")
    f = pl.kernel(
        _pass_a_body,
        mesh=mesh,
        compiler_params=pltpu.CompilerParams(use_tc_tiling_on_sc=False, needs_layout_passes=False),
        out_type=[
            jax.ShapeDtypeStruct((NW, 16), jnp.float32),
            jax.ShapeDtypeStruct((NW, 16), jnp.float32),
        ],
        scratch_types=[
            pltpu.VMEM((WA,), jnp.int32),
            pltpu.VMEM((WA,), jnp.int32),
            pltpu.VMEM((WA, XPAD), jnp.float32),
            pltpu.VMEM((WA, XPAD), jnp.float32),
            pltpu.VMEM((32,), jnp.float32),
            pltpu.SemaphoreType.DMA,
            pltpu.SemaphoreType.DMA,
        ],
    )
    return f(xp, idxi, idxj)


def _pass_b_body(u, v, idxi, idxj, hpre, st_out, bi, bj, ru, rv, hb, acc, sem1, sem2):
    w = _wid()
    base = w * EPT

    def window(wi, carry):
        off = base + wi * WA
        pltpu.sync_copy(idxi.at[pl.ds(off, WA)], bi)
        pltpu.sync_copy(idxj.at[pl.ds(off, WA)], bj)
        cp1 = pltpu.async_copy(u.at[bi], ru, sem1)
        cp2 = pltpu.async_copy(v.at[bj], rv, sem2)
        cp1.wait()
        cp2.wait()

        def edge(e, carry2):
            out = []
            for c8 in range(8):
                h = ru[e, pl.ds(c8 * 16, 16)] + rv[e, pl.ds(c8 * 16, 16)]
                hb[e, pl.ds(c8 * 16, 16)] = h
                s1 = carry2[2 * c8] + h
                s2 = carry2[2 * c8 + 1] + h * h
                out.append(s1)
                out.append(s2)
            return tuple(out)

        carry = lax.fori_loop(0, WA, edge, carry)
        pltpu.sync_copy(hb, hpre.at[pl.ds(off, WA)])
        return carry

    z = jnp.zeros((16,), jnp.float32)
    carry = tuple(z for _ in range(16))
    carry = lax.fori_loop(0, EPT // WA, window, carry)
    for c8 in range(8):
        acc[pl.ds(c8 * 16, 16)] = carry[2 * c8]
        acc[pl.ds(128 + c8 * 16, 16)] = carry[2 * c8 + 1]
    pltpu.sync_copy(acc, st_out.at[w])


def _run_pass_b(u, v, idxi, idxj):
    mesh = plsc.VectorSubcoreMesh(core_axis_name="c", subcore_axis_name="s")
    f = pl.kernel(
        _pass_b_body,
        mesh=mesh,
        out_type=[
            jax.ShapeDtypeStruct((E, C), jnp.float32),
            jax.ShapeDtypeStruct((NW, 2 * C), jnp.float32),
        ],
        scratch_types=[
            pltpu.VMEM((WA,), jnp.int32),
            pltpu.VMEM((WA,), jnp.int32),
            pltpu.VMEM((WA, C), jnp.float32),
            pltpu.VMEM((WA, C), jnp.float32),
            pltpu.VMEM((WA, C), jnp.float32),
            pltpu.VMEM((2 * C,), jnp.float32),
            pltpu.SemaphoreType.DMA,
            pltpu.SemaphoreType.DMA,
        ],
    )
    return f(u, v, idxi, idxj)


def _pass_h_body(h2, idxi, out, iba, ibb, sel_id, sel_dst, rows, mx, sm, cnt,
                 sema, semb, semr):
    w = _wid()
    lo = w * NPT
    hi = lo + NPT
    zero = jnp.zeros((16,), jnp.float32)
    one = jnp.ones((16,), jnp.float32)

    def zinit(n, _):
        for c8 in range(8):
            mx[n, pl.ds(c8 * 16, 16)] = zero
            sm[n, pl.ds(c8 * 16, 16)] = zero
        cnt[n, pl.ds(0, 16)] = zero
        return 0

    lax.fori_loop(0, NPT, zinit, 0)

    izero = jnp.zeros((16,), jnp.int32)

    def selinit(k, _):
        sel_id[pl.ds(k * 16, 16)] = izero
        sel_dst[pl.ds(k * 16, 16)] = izero
        return 0

    lax.fori_loop(0, SELB // 16, selinit, 0)

    def process(ib, off):
        def grp(g, fill):
            vv = ib[pl.ds(g * 16, 16)]
            m = (vv >= lo) & (vv < hi)
            ids = lax.iota(jnp.int32, 16) + (off + g * 16)
            pref = plsc.cumsum(jnp.where(m, 1, 0))
            pos = fill + pref - 1
            plsc.store_scatter(sel_id, [pos], ids, mask=m)
            plsc.store_scatter(sel_dst, [pos], vv, mask=m)
            return fill + pref[15]

        fill = lax.fori_loop(0, WH // 16, grp, jnp.int32(0))

        nch = (fill + (BH - 1)) // BH

        def chunk(ci, _):
            cb = ci * BH
            idvec = sel_id[pl.ds(cb, BH)]
            pltpu.async_copy(h2.at[idvec], rows, semr).wait()
            nk = jnp.minimum(fill - cb, BH)

            def edge(k, _2):
                d = sel_dst[pl.ds(cb + k, 16)][0]
                dl = d - lo
                for c8 in range(8):
                    r = rows[k, pl.ds(c8 * 16, 16)]
                    a = mx[dl, pl.ds(c8 * 16, 16)]
                    mx[dl, pl.ds(c8 * 16, 16)] = jnp.maximum(a, r)
                    s = sm[dl, pl.ds(c8 * 16, 16)]
                    sm[dl, pl.ds(c8 * 16, 16)] = s + r
                cnt[dl, pl.ds(0, 16)] = cnt[dl, pl.ds(0, 16)] + one
                return 0

            lax.fori_loop(0, nk, edge, 0)
            return 0

        lax.fori_loop(0, nch, chunk, 0)

    pltpu.async_copy(idxi.at[pl.ds(0, WH)], iba, sema)

    def pair(pi, _):
        w0 = 2 * pi
        pltpu.async_copy(idxi.at[pl.ds((w0 + 1) * WH, WH)], ibb, semb)
        pltpu.make_async_copy(idxi.at[pl.ds(0, WH)], iba, sema).wait()
        process(iba, w0 * WH)
        nxt = jnp.minimum(w0 + 2, NWIN - 1)
        pltpu.async_copy(idxi.at[pl.ds(nxt * WH, WH)], iba, sema)
        pltpu.make_async_copy(idxi.at[pl.ds(0, WH)], ibb, semb).wait()
        process(ibb, (w0 + 1) * WH)
        return 0

    lax.fori_loop(0, NWIN // 2, pair, 0)
    pltpu.make_async_copy(idxi.at[pl.ds(0, WH)], iba, sema).wait()

    def finish(n, _):
        cv = cnt[n, pl.ds(0, 16)]
        den = jnp.maximum(cv, 1.0)
        for c8 in range(8):
            s = sm[n, pl.ds(c8 * 16, 16)]
            m = mx[n, pl.ds(c8 * 16, 16)]
            mx[n, pl.ds(c8 * 16, 16)] = m + s / den
        return 0

    lax.fori_loop(0, NPT, finish, 0)
    pltpu.sync_copy(mx, out.at[pl.ds(lo, NPT)])


def _run_pass_h(h2, idxi):
    mesh = plsc.VectorSubcoreMesh(core_axis_name="c", subcore_axis_name="s")
    f = pl.kernel(
        _pass_h_body,
        mesh=mesh,
        compiler_params=pltpu.CompilerParams(needs_layout_passes=False),
        out_type=jax.ShapeDtypeStruct((NPAD, C), jnp.float32),
        scratch_types=[
            pltpu.VMEM((WH,), jnp.int32),
            pltpu.VMEM((WH,), jnp.int32),
            pltpu.VMEM((SELB,), jnp.int32),
            pltpu.VMEM((SELB,), jnp.int32),
            pltpu.VMEM((BH, C), jnp.float32),
            pltpu.VMEM((NPT, C), jnp.float32),
            pltpu.VMEM((NPT, C), jnp.float32),
            pltpu.VMEM((NPT, 16), jnp.float32),
            pltpu.SemaphoreType.DMA,
            pltpu.SemaphoreType.DMA,
            pltpu.SemaphoreType.DMA,
        ],
    )
    return f(h2, idxi)


TN = 640
TE = 4000


def _node_body(xp_ref, w_ref, cvec_ref, inv_ref, u_ref, v_ref):
    z = jnp.dot(xp_ref[...], w_ref[...], preferred_element_type=jnp.float32)
    zb = z[:, C:] * inv_ref[0, 0]
    u_ref[...] = z[:, :C] + cvec_ref[...] - zb
    v_ref[...] = zb


def _run_node(xp_pad, wcat, cvec, inv):
    grid = (NPAD // TN,)
    return pl.pallas_call(
        _node_body,
        grid=grid,
        in_specs=[
            pl.BlockSpec((TN, XPAD), lambda i: (i, 0)),
            pl.BlockSpec((XPAD, 2 * C), lambda i: (0, 0)),
            pl.BlockSpec((1, C), lambda i: (0, 0)),
            pl.BlockSpec((1, 1), lambda i: (0, 0), memory_space=pltpu.SMEM),
        ],
        out_specs=[
            pl.BlockSpec((TN, C), lambda i: (i, 0)),
            pl.BlockSpec((TN, C), lambda i: (i, 0)),
        ],
        out_shape=[
            jax.ShapeDtypeStruct((NPAD, C), jnp.float32),
            jax.ShapeDtypeStruct((NPAD, C), jnp.float32),
        ],
    )(xp_pad, wcat, cvec, inv)


def _lin_body(x_ref, a_ref, b_ref, w_ref, bias_ref, out_ref, st_ref, acc_ref):
    i = pl.program_id(0)
    t = jnp.maximum(x_ref[...] * a_ref[...] + b_ref[...], 0.0)
    m = jnp.dot(t, w_ref[...], preferred_element_type=jnp.float32) + bias_ref[...]
    out_ref[...] = m

    @pl.when(i == 0)
    def _():
        acc_ref[...] = jnp.zeros_like(acc_ref)

    acc_ref[0:1, :] += jnp.sum(m, axis=0, keepdims=True)
    acc_ref[1:2, :] += jnp.sum(m * m, axis=0, keepdims=True)

    @pl.when(i == pl.num_programs(0) - 1)
    def _():
        st_ref[...] = acc_ref[...]


def _run_lin(x, a, b, w, bias):
    grid = (E // TE,)
    return pl.pallas_call(
        _lin_body,
        grid=grid,
        in_specs=[
            pl.BlockSpec((TE, C), lambda i: (i, 0)),
            pl.BlockSpec((1, C), lambda i: (0, 0)),
            pl.BlockSpec((1, C), lambda i: (0, 0)),
            pl.BlockSpec((C, C), lambda i: (0, 0)),
            pl.BlockSpec((1, C), lambda i: (0, 0)),
        ],
        out_specs=[
            pl.BlockSpec((TE, C), lambda i: (i, 0)),
            pl.BlockSpec((2, C), lambda i: (0, 0)),
        ],
        out_shape=[
            jax.ShapeDtypeStruct((E, C), jnp.float32),
            jax.ShapeDtypeStruct((2, C), jnp.float32),
        ],
        scratch_shapes=[pltpu.VMEM((2, C), jnp.float32)],
    )(x, a, b, w, bias)


def _lin_res_body(x_ref, r_ref, a_ref, b_ref, ar_ref, br_ref, w_ref, bias_ref,
                  out_ref, st_ref, acc_ref):
    i = pl.program_id(0)
    h0 = jnp.maximum(r_ref[...] * ar_ref[...] + br_ref[...], 0.0)
    t = jnp.maximum(x_ref[...] * a_ref[...] + b_ref[...] + h0, 0.0)
    m = jnp.dot(t, w_ref[...], preferred_element_type=jnp.float32) + bias_ref[...]
    out_ref[...] = m

    @pl.when(i == 0)
    def _():
        acc_ref[...] = jnp.zeros_like(acc_ref)

    acc_ref[0:1, :] += jnp.sum(m, axis=0, keepdims=True)
    acc_ref[1:2, :] += jnp.sum(m * m, axis=0, keepdims=True)

    @pl.when(i == pl.num_programs(0) - 1)
    def _():
        st_ref[...] = acc_ref[...]


def _run_lin_res(x, r, a, b, ar, br, w, bias):
    grid = (E // TE,)
    return pl.pallas_call(
        _lin_res_body,
        grid=grid,
        in_specs=[
            pl.BlockSpec((TE, C), lambda i: (i, 0)),
            pl.BlockSpec((TE, C), lambda i: (i, 0)),
            pl.BlockSpec((1, C), lambda i: (0, 0)),
            pl.BlockSpec((1, C), lambda i: (0, 0)),
            pl.BlockSpec((1, C), lambda i: (0, 0)),
            pl.BlockSpec((1, C), lambda i: (0, 0)),
            pl.BlockSpec((C, C), lambda i: (0, 0)),
            pl.BlockSpec((1, C), lambda i: (0, 0)),
        ],
        out_specs=[
            pl.BlockSpec((TE, C), lambda i: (i, 0)),
            pl.BlockSpec((2, C), lambda i: (0, 0)),
        ],
        out_shape=[
            jax.ShapeDtypeStruct((E, C), jnp.float32),
            jax.ShapeDtypeStruct((2, C), jnp.float32),
        ],
        scratch_shapes=[pltpu.VMEM((2, C), jnp.float32)],
    )(x, r, a, b, ar, br, w, bias)


def _final_body(m4_ref, m2_ref, hp_ref, a4_ref, b4_ref, a2_ref, b2_ref,
                a0_ref, b0_ref, out_ref):
    h0 = jnp.maximum(hp_ref[...] * a0_ref[...] + b0_ref[...], 0.0)
    h1 = jnp.maximum(m2_ref[...] * a2_ref[...] + b2_ref[...] + h0, 0.0)
    out_ref[...] = jnp.maximum(m4_ref[...] * a4_ref[...] + b4_ref[...] + h1, 0.0)


def _run_final(m4, m2, hp, a4, b4, a2, b2, a0, b0):
    grid = (E // TE,)

    def vec():
        return pl.BlockSpec((1, C), lambda i: (0, 0))

    def big():
        return pl.BlockSpec((TE, C), lambda i: (i, 0))

    return pl.pallas_call(
        _final_body,
        grid=grid,
        in_specs=[big(), big(), big(), vec(), vec(), vec(), vec(), vec(), vec()],
        out_specs=big(),
        out_shape=jax.ShapeDtypeStruct((E, C), jnp.float32),
    )(m4, m2, hp, a4, b4, a2, b2, a0, b0)


def _bn_affine(st, g, beta):
    mu = st[0] / E
    var = st[1] / E - mu * mu
    a = g / jnp.sqrt(var + 1e-5)
    return (a[None, :], (beta - mu * a)[None, :])


def kernel(pos, x, edge_index, affine_w, affine_b, lin_W, lin_b, lin_g, lin_beta,
           rW1, rb1, rg1, rbe1, rW2, rb2, rgn, rbn):
    idx_i = edge_index[0]
    idx_j = edge_index[1]

    xf = jnp.concatenate([x, pos], axis=1)
    xp = jnp.pad(xf, ((0, 0), (0, XPAD - CP)))

    s1p, s2p = _run_pass_a(xp, idx_i, idx_j)
    s1 = jnp.sum(s1p)
    s2 = jnp.sum(s2p)
    m = E * CP
    var = (s2 - s1 * s1 / m) / (m - 1)
    sprime = jnp.sqrt(var) + 1e-5

    w_top = lin_W[:CP]
    w_bot = affine_w[:, None] * lin_W[CP:]
    wcat = jnp.pad(jnp.concatenate([w_top, w_bot], axis=1),
                   ((0, XPAD - CP), (0, 0)))
    c0 = affine_b @ lin_W[CP:] + lin_b
    xp_pad = jnp.pad(xp, ((0, NPAD - N), (0, 0)))
    inv = (1.0 / sprime).reshape(1, 1)
    u, v = _run_node(xp_pad, wcat, c0[None, :], inv)

    hpre, stp = _run_pass_b(u, v, idx_i, idx_j)
    st0 = jnp.sum(stp, axis=0)
    a0, b0 = _bn_affine(jnp.stack([st0[:C], st0[C:]]), lin_g, lin_beta)

    m1, st1 = _run_lin(hpre, a0, b0, rW1[0], rb1[0][None, :])
    a1, b1 = _bn_affine(st1, rg1[0], rbe1[0])
    m2, st2 = _run_lin(m1, a1, b1, rW2[0], rb2[0][None, :])
    a2, b2 = _bn_affine(st2, rgn[0], rbn[0])
    m3, st3 = _run_lin_res(m2, hpre, a2, b2, a0, b0, rW1[1], rb1[1][None, :])
    a3, b3 = _bn_affine(st3, rg1[1], rbe1[1])
    m4, st4 = _run_lin(m3, a3, b3, rW2[1], rb2[1][None, :])
    a4, b4 = _bn_affine(st4, rgn[1], rbn[1])
    h2 = _run_final(m4, m2, hpre, a4, b4, a2, b2, a0, b0)

    out = _run_pass_h(h2, idx_i)
    return out[:N]
